# Initial kernel scaffold; baseline (speedup 1.0000x reference)
import functools

import jax
import jax.numpy as jnp
from jax import lax
from jax.experimental import pallas as pl
from jax.experimental.pallas import tpu as pltpu
from jax.experimental.pallas import tpu_sc as plsc

_N = 50000
_E = 800000
_R = 16
_L = 2
_H = 3 * _R
_DEMB = 16
_NSC = 16
_DROT = 2 * _L * _R
_DOUT = 2 * _R

_NC = 2
_NS = 16
_NW = _NC * _NS

_CHUNK = 128
_G_CHUNKS = _E // _CHUNK
_G_ITERS = -(-_G_CHUNKS // _NW)
_E_PER_CORE = _E // _NC
_S_CHUNKS = _E_PER_CORE // _CHUNK
_S_ITERS = -(-_S_CHUNKS // _NS)
_ROWS_PER_TILE = _N // _NS

_mesh = plsc.VectorSubcoreMesh(core_axis_name="c", subcore_axis_name="s")


@functools.partial(
    pl.kernel,
    out_type=(
        jax.ShapeDtypeStruct((_E, _DROT), jnp.float32),
        jax.ShapeDtypeStruct((_E, _NSC), jnp.float32),
    ),
    mesh=_mesh,
    scratch_types=[
        pltpu.VMEM((_CHUNK,), jnp.int32),
        pltpu.VMEM((_CHUNK, _DROT), jnp.float32),
        pltpu.VMEM((_CHUNK, _NSC), jnp.float32),
        pltpu.SemaphoreType.DMA,
        pltpu.SemaphoreType.DMA,
    ],
)
def _gather(row_hbm, xrot_hbm, xsc_hbm, orot_hbm, osc_hbm,
            idx_v, rot_v, sc_v, sem1, sem2):
    wid = lax.axis_index("s") * _NC + lax.axis_index("c")

    def body(j, carry):
        cid = j * _NW + wid

        @pl.when(cid < _G_CHUNKS)
        def _():
            base = cid * _CHUNK
            pltpu.sync_copy(row_hbm.at[pl.ds(base, _CHUNK)], idx_v)
            a = pltpu.async_copy(xrot_hbm.at[idx_v], rot_v, sem1)
            b = pltpu.async_copy(xsc_hbm.at[idx_v], sc_v, sem2)
            a.wait()
            b.wait()
            pltpu.sync_copy(rot_v, orot_hbm.at[pl.ds(base, _CHUNK)])
            pltpu.sync_copy(sc_v, osc_hbm.at[pl.ds(base, _CHUNK)])

        return carry

    lax.fori_loop(0, _G_ITERS, body, 0)


@functools.partial(
    pl.kernel,
    out_type=jax.ShapeDtypeStruct((_NC, _N, _DOUT), jnp.float32),
    mesh=_mesh,
    scratch_types=[
        pltpu.VMEM((_CHUNK,), jnp.int32),
        pltpu.VMEM((_CHUNK, _DOUT), jnp.float32),
        pltpu.VMEM_SHARED((_N, _DOUT), jnp.float32),
    ],
)
def _scatter(col_hbm, xout_hbm, zero_hbm, part_hbm, idx_v, rows_v, acc):
    c = lax.axis_index("c")
    s = lax.axis_index("s")
    r0 = s * _ROWS_PER_TILE
    pltpu.sync_copy(zero_hbm.at[pl.ds(r0, _ROWS_PER_TILE)],
                    acc.at[pl.ds(r0, _ROWS_PER_TILE)])
    plsc.subcore_barrier()

    def body(j, carry):
        lcid = j * _NS + s

        @pl.when(lcid < _S_CHUNKS)
        def _():
            base = c * _E_PER_CORE + lcid * _CHUNK
            pltpu.sync_copy(col_hbm.at[pl.ds(base, _CHUNK)], idx_v)
            pltpu.sync_copy(xout_hbm.at[pl.ds(base, _CHUNK)], rows_v)
            pltpu.sync_copy(rows_v, acc.at[idx_v], add=True)

        return carry

    lax.fori_loop(0, _S_ITERS, body, 0)
    plsc.subcore_barrier()
    pltpu.sync_copy(acc.at[pl.ds(r0, _ROWS_PER_TILE)],
                    part_hbm.at[c, pl.ds(r0, _ROWS_PER_TILE)])


_BE = 4000


def _mlp_body(grot_ref, gsc_ref, demb_ref, wd_ref, wx_ref, b_ref,
              mrot_ref, m2_ref, out_ref):
    z = (jnp.dot(demb_ref[...], wd_ref[...], preferred_element_type=jnp.float32)
         + jnp.dot(gsc_ref[...], wx_ref[...], preferred_element_type=jnp.float32)
         + b_ref[...])
    g = z * jax.nn.sigmoid(z)
    h2 = jnp.dot(grot_ref[...], mrot_ref[...], preferred_element_type=jnp.float32)
    out_ref[...] = jnp.dot(h2 * g, m2_ref[...], preferred_element_type=jnp.float32)


_mlp = pl.pallas_call(
    _mlp_body,
    grid=(_E // _BE,),
    in_specs=[
        pl.BlockSpec((_BE, _DROT), lambda i: (i, 0)),
        pl.BlockSpec((_BE, _NSC), lambda i: (i, 0)),
        pl.BlockSpec((_BE, _DEMB), lambda i: (i, 0)),
        pl.BlockSpec((_DEMB, 4 * _H), lambda i: (0, 0)),
        pl.BlockSpec((_NSC, 4 * _H), lambda i: (0, 0)),
        pl.BlockSpec((1, 4 * _H), lambda i: (0, 0)),
        pl.BlockSpec((_DROT, 4 * _H), lambda i: (0, 0)),
        pl.BlockSpec((4 * _H, _DOUT), lambda i: (0, 0)),
    ],
    out_specs=pl.BlockSpec((_BE, _DOUT), lambda i: (i, 0)),
    out_shape=jax.ShapeDtypeStruct((_E, _DOUT), jnp.float32),
)


def _add_body(a_ref, b_ref, o_ref):
    o_ref[...] = a_ref[...] + b_ref[...]


_BN = 2500

_combine = pl.pallas_call(
    _add_body,
    grid=(_N // _BN,),
    in_specs=[
        pl.BlockSpec((_BN, _DOUT), lambda i: (i, 0)),
        pl.BlockSpec((_BN, _DOUT), lambda i: (i, 0)),
    ],
    out_specs=pl.BlockSpec((_BN, _DOUT), lambda i: (i, 0)),
    out_shape=jax.ShapeDtypeStruct((_N, _DOUT), jnp.float32),
)


def kernel(x_scalar, x_rot, edge_index, distance_embedding, rot,
           W_rot, W_s1, b_s1, W_out):
    del rot
    row = edge_index[0]
    col = edge_index[1]
    xr2 = x_rot.reshape(_N, _DROT)

    grot, gsc = _gather(row, xr2, x_scalar)

    mrot = jnp.kron(W_rot, jnp.eye(4, dtype=jnp.float32))
    ws1e = jnp.broadcast_to(
        W_s1.reshape(_DEMB + _NSC, _H, _L, 1),
        (_DEMB + _NSC, _H, _L, 2)).reshape(_DEMB + _NSC, 4 * _H)
    wd = ws1e[:_DEMB]
    wx = ws1e[_DEMB:]
    be = jnp.broadcast_to(b_s1.reshape(_H, _L, 1),
                          (_H, _L, 2)).reshape(1, 4 * _H)
    m2 = jnp.einsum("hlo,cd->hlcod", W_out,
                    jnp.eye(2, dtype=jnp.float32)).reshape(4 * _H, _DOUT)

    xout = _mlp(grot, gsc, distance_embedding, wd, wx, be, mrot, m2)

    zeros = jnp.zeros((_N, _DOUT), jnp.float32)
    parts = _scatter(col, xout, zeros)
    mess = _combine(parts[0], parts[1])
    return (x_scalar, mess.reshape(_N, _R, 2))


# speedup vs baseline: 3.7203x; 3.7203x over previous
import functools

import jax
import jax.numpy as jnp
from jax import lax
from jax.experimental import pallas as pl
from jax.experimental.pallas import tpu as pltpu
from jax.experimental.pallas import tpu_sc as plsc

_N = 50000
_E = 800000
_R = 16
_L = 2
_H = 3 * _R
_DEMB = 16
_NSC = 16
_DROT = 2 * _L * _R
_DOUT = 2 * _R

_NC = 2
_NS = 16
_NW = _NC * _NS

_CHUNK = 128
_G_CHUNKS = _E // _CHUNK
_G_ITERS = -(-_G_CHUNKS // _NW)
_E_PER_CORE = _E // _NC
_S_CHUNKS = _E_PER_CORE // _CHUNK
_S_ITERS = -(-_S_CHUNKS // _NS)
_ROWS_PER_TILE = _N // _NS

_mesh = plsc.VectorSubcoreMesh(core_axis_name="c", subcore_axis_name="s")


@functools.partial(
    pl.kernel,
    out_type=(
        jax.ShapeDtypeStruct((_E, _DROT), jnp.float32),
        jax.ShapeDtypeStruct((_E, _NSC), jnp.float32),
    ),
    mesh=_mesh,
    scratch_types=[
        pltpu.VMEM((_CHUNK,), jnp.int32),
        pltpu.VMEM((_CHUNK, _DROT), jnp.float32),
        pltpu.VMEM((_CHUNK, _NSC), jnp.float32),
        pltpu.SemaphoreType.DMA,
        pltpu.SemaphoreType.DMA,
    ],
    compiler_params=pltpu.CompilerParams(use_tc_tiling_on_sc=False),
)
def _gather(row_hbm, xrot_hbm, xsc_hbm, orot_hbm, osc_hbm,
            idx_v, rot_v, sc_v, sem1, sem2):
    wid = lax.axis_index("s") * _NC + lax.axis_index("c")

    def body(j, carry):
        cid = j * _NW + wid

        @pl.when(cid < _G_CHUNKS)
        def _():
            base = cid * _CHUNK
            pltpu.sync_copy(row_hbm.at[pl.ds(base, _CHUNK)], idx_v)
            a = pltpu.async_copy(xrot_hbm.at[idx_v], rot_v, sem1)
            b = pltpu.async_copy(xsc_hbm.at[idx_v], sc_v, sem2)
            a.wait()
            b.wait()
            pltpu.sync_copy(rot_v, orot_hbm.at[pl.ds(base, _CHUNK)])
            pltpu.sync_copy(sc_v, osc_hbm.at[pl.ds(base, _CHUNK)])

        return carry

    lax.fori_loop(0, _G_ITERS, body, 0)


@functools.partial(
    pl.kernel,
    out_type=jax.ShapeDtypeStruct((_NC, _N, _DOUT), jnp.float32),
    mesh=_mesh,
    scratch_types=[
        pltpu.VMEM((_CHUNK,), jnp.int32),
        pltpu.VMEM((_CHUNK, _DOUT), jnp.float32),
        pltpu.VMEM_SHARED((_N, _DOUT), jnp.float32),
    ],
    compiler_params=pltpu.CompilerParams(use_tc_tiling_on_sc=False),
)
def _scatter(col_hbm, xout_hbm, zero_hbm, part_hbm, idx_v, rows_v, acc):
    c = lax.axis_index("c")
    s = lax.axis_index("s")
    r0 = s * _ROWS_PER_TILE
    pltpu.sync_copy(zero_hbm.at[pl.ds(r0, _ROWS_PER_TILE)],
                    acc.at[pl.ds(r0, _ROWS_PER_TILE)])
    plsc.subcore_barrier()

    def body(j, carry):
        lcid = j * _NS + s

        @pl.when(lcid < _S_CHUNKS)
        def _():
            base = c * _E_PER_CORE + lcid * _CHUNK
            pltpu.sync_copy(col_hbm.at[pl.ds(base, _CHUNK)], idx_v)
            pltpu.sync_copy(xout_hbm.at[pl.ds(base, _CHUNK)], rows_v)
            pltpu.sync_copy(rows_v, acc.at[idx_v], add=True)

        return carry

    lax.fori_loop(0, _S_ITERS, body, 0)
    plsc.subcore_barrier()
    pltpu.sync_copy(acc.at[pl.ds(r0, _ROWS_PER_TILE)],
                    part_hbm.at[c, pl.ds(r0, _ROWS_PER_TILE)])


_BE = 4000


def _mlp_body(grot_ref, gsc_ref, demb_ref, wd_ref, wx_ref, b_ref,
              mrot_ref, m2_ref, out_ref):
    z = (jnp.dot(demb_ref[...], wd_ref[...], preferred_element_type=jnp.float32)
         + jnp.dot(gsc_ref[...], wx_ref[...], preferred_element_type=jnp.float32)
         + b_ref[...])
    g = z * jax.nn.sigmoid(z)
    h2 = jnp.dot(grot_ref[...], mrot_ref[...], preferred_element_type=jnp.float32)
    out_ref[...] = jnp.dot(h2 * g, m2_ref[...], preferred_element_type=jnp.float32)


_mlp = pl.pallas_call(
    _mlp_body,
    grid=(_E // _BE,),
    in_specs=[
        pl.BlockSpec((_BE, _DROT), lambda i: (i, 0)),
        pl.BlockSpec((_BE, _NSC), lambda i: (i, 0)),
        pl.BlockSpec((_BE, _DEMB), lambda i: (i, 0)),
        pl.BlockSpec((_DEMB, 4 * _H), lambda i: (0, 0)),
        pl.BlockSpec((_NSC, 4 * _H), lambda i: (0, 0)),
        pl.BlockSpec((1, 4 * _H), lambda i: (0, 0)),
        pl.BlockSpec((_DROT, 4 * _H), lambda i: (0, 0)),
        pl.BlockSpec((4 * _H, _DOUT), lambda i: (0, 0)),
    ],
    out_specs=pl.BlockSpec((_BE, _DOUT), lambda i: (i, 0)),
    out_shape=jax.ShapeDtypeStruct((_E, _DOUT), jnp.float32),
)


def _add_body(a_ref, b_ref, o_ref):
    o_ref[...] = a_ref[...] + b_ref[...]


_BN = 2000

_combine = pl.pallas_call(
    _add_body,
    grid=(_N // _BN,),
    in_specs=[
        pl.BlockSpec((_BN, _DOUT), lambda i: (i, 0)),
        pl.BlockSpec((_BN, _DOUT), lambda i: (i, 0)),
    ],
    out_specs=pl.BlockSpec((_BN, _DOUT), lambda i: (i, 0)),
    out_shape=jax.ShapeDtypeStruct((_N, _DOUT), jnp.float32),
)


def kernel(x_scalar, x_rot, edge_index, distance_embedding, rot,
           W_rot, W_s1, b_s1, W_out):
    del rot
    row = edge_index[0]
    col = edge_index[1]
    xr2 = x_rot.reshape(_N, _DROT)

    grot, gsc = _gather(row, xr2, x_scalar)

    mrot = jnp.kron(W_rot, jnp.eye(4, dtype=jnp.float32))
    ws1e = jnp.broadcast_to(
        W_s1.reshape(_DEMB + _NSC, _H, _L, 1),
        (_DEMB + _NSC, _H, _L, 2)).reshape(_DEMB + _NSC, 4 * _H)
    wd = ws1e[:_DEMB]
    wx = ws1e[_DEMB:]
    be = jnp.broadcast_to(b_s1.reshape(_H, _L, 1),
                          (_H, _L, 2)).reshape(1, 4 * _H)
    m2 = jnp.einsum("hlo,cd->hlcod", W_out,
                    jnp.eye(2, dtype=jnp.float32)).reshape(4 * _H, _DOUT)

    xout = _mlp(grot, gsc, distance_embedding, wd, wx, be, mrot, m2)

    zeros = jnp.zeros((_N, _DOUT), jnp.float32)
    parts = _scatter(col, xout, zeros)
    mess = _combine(parts[0], parts[1])
    return (x_scalar, mess.reshape(_N, _R, 2))


# speedup vs baseline: 5.0515x; 1.3578x over previous
import functools

import jax
import jax.numpy as jnp
from jax import lax
from jax.experimental import pallas as pl
from jax.experimental.pallas import tpu as pltpu
from jax.experimental.pallas import tpu_sc as plsc

_N = 50000
_E = 800000
_R = 16
_L = 2
_H = 3 * _R
_DEMB = 16
_NSC = 16
_DROT = 2 * _L * _R
_DOUT = 2 * _R
_DT = 128

_NC = 2
_NS = 16
_NW = _NC * _NS

_CHUNK = 128
_G_CHUNKS = _E // _CHUNK
_G_ITERS = -(-_G_CHUNKS // _NW)
_E_PER_CORE = _E // _NC
_S_CHUNKS = _E_PER_CORE // _CHUNK
_S_ITERS = -(-_S_CHUNKS // _NS)
_ROWS_PER_TILE = _N // _NS

_mesh = plsc.VectorSubcoreMesh(core_axis_name="c", subcore_axis_name="s")


@functools.partial(
    pl.kernel,
    out_type=jax.ShapeDtypeStruct((_E, _DT), jnp.float32),
    mesh=_mesh,
    scratch_types=[
        pltpu.VMEM((_CHUNK,), jnp.int32),
        pltpu.VMEM((_CHUNK, _DT), jnp.float32),
        pltpu.SemaphoreType.DMA,
    ],
)
def _gather(row_hbm, tbl_hbm, out_hbm, idx_v, rows_v, sem):
    wid = lax.axis_index("s") * _NC + lax.axis_index("c")

    def body(j, carry):
        cid = j * _NW + wid

        @pl.when(cid < _G_CHUNKS)
        def _():
            base = cid * _CHUNK
            pltpu.sync_copy(row_hbm.at[pl.ds(base, _CHUNK)], idx_v)
            pltpu.async_copy(tbl_hbm.at[idx_v], rows_v, sem).wait()
            pltpu.sync_copy(rows_v, out_hbm.at[pl.ds(base, _CHUNK)])

        return carry

    lax.fori_loop(0, _G_ITERS, body, 0)


@functools.partial(
    pl.kernel,
    out_type=jax.ShapeDtypeStruct((_NC, _N, _DOUT), jnp.float32),
    mesh=_mesh,
    scratch_types=[
        pltpu.VMEM((_CHUNK,), jnp.int32),
        pltpu.VMEM((_CHUNK, _DOUT), jnp.float32),
        pltpu.VMEM_SHARED((_N, _DOUT), jnp.float32),
    ],
    compiler_params=pltpu.CompilerParams(use_tc_tiling_on_sc=False),
)
def _scatter(col_hbm, xout_hbm, zero_hbm, part_hbm, idx_v, rows_v, acc):
    c = lax.axis_index("c")
    s = lax.axis_index("s")
    r0 = s * _ROWS_PER_TILE
    pltpu.sync_copy(zero_hbm.at[pl.ds(r0, _ROWS_PER_TILE)],
                    acc.at[pl.ds(r0, _ROWS_PER_TILE)])
    plsc.subcore_barrier()

    def body(j, carry):
        lcid = j * _NS + s

        @pl.when(lcid < _S_CHUNKS)
        def _():
            base = c * _E_PER_CORE + lcid * _CHUNK
            pltpu.sync_copy(col_hbm.at[pl.ds(base, _CHUNK)], idx_v)
            pltpu.sync_copy(xout_hbm.at[pl.ds(base, _CHUNK)], rows_v)
            pltpu.sync_copy(rows_v, acc.at[idx_v], add=True)

        return carry

    lax.fori_loop(0, _S_ITERS, body, 0)
    plsc.subcore_barrier()
    pltpu.sync_copy(acc.at[pl.ds(r0, _ROWS_PER_TILE)],
                    part_hbm.at[c, pl.ds(r0, _ROWS_PER_TILE)])


_BE = 4000


def _mlp_body(gthr_ref, demb_ref, wd_ref, wx_ref, b_ref,
              mrot_ref, m2_ref, out_ref):
    gt = gthr_ref[...]
    xr = gt[:, :_DROT]
    xs = gt[:, _DROT:_DROT + _NSC]
    z = (jnp.dot(demb_ref[...], wd_ref[...], preferred_element_type=jnp.float32)
         + jnp.dot(xs, wx_ref[...], preferred_element_type=jnp.float32)
         + b_ref[...])
    g = z * jax.nn.sigmoid(z)
    h2 = jnp.dot(xr, mrot_ref[...], preferred_element_type=jnp.float32)
    res = jnp.dot(h2 * g, m2_ref[...], preferred_element_type=jnp.float32)
    q = _BE // 4
    out_ref[...] = jnp.concatenate(
        [res[0:q], res[q:2 * q], res[2 * q:3 * q], res[3 * q:4 * q]], axis=1)


_mlp = pl.pallas_call(
    _mlp_body,
    grid=(_E // _BE,),
    in_specs=[
        pl.BlockSpec((_BE, _DT), lambda i: (i, 0)),
        pl.BlockSpec((_BE, _DEMB), lambda i: (i, 0)),
        pl.BlockSpec((_DEMB, 4 * _H), lambda i: (0, 0)),
        pl.BlockSpec((_NSC, 4 * _H), lambda i: (0, 0)),
        pl.BlockSpec((1, 4 * _H), lambda i: (0, 0)),
        pl.BlockSpec((_DROT, 4 * _H), lambda i: (0, 0)),
        pl.BlockSpec((4 * _H, _DOUT), lambda i: (0, 0)),
    ],
    out_specs=pl.BlockSpec((_BE // 4, 128), lambda i: (i, 0)),
    out_shape=jax.ShapeDtypeStruct((_E // 4, 128), jnp.float32),
)


def _add_body(a_ref, b_ref, o_ref):
    o_ref[...] = a_ref[...] + b_ref[...]


_BN = 2000

_combine = pl.pallas_call(
    _add_body,
    grid=(_N // _BN,),
    in_specs=[
        pl.BlockSpec((_BN, _DOUT), lambda i: (i, 0)),
        pl.BlockSpec((_BN, _DOUT), lambda i: (i, 0)),
    ],
    out_specs=pl.BlockSpec((_BN, _DOUT), lambda i: (i, 0)),
    out_shape=jax.ShapeDtypeStruct((_N, _DOUT), jnp.float32),
)


def kernel(x_scalar, x_rot, edge_index, distance_embedding, rot,
           W_rot, W_s1, b_s1, W_out):
    del rot
    row = edge_index[0]
    col = edge_index[1]
    tbl = jnp.concatenate(
        [x_rot.reshape(_N, _DROT), x_scalar,
         jnp.zeros((_N, _DT - _DROT - _NSC), jnp.float32)], axis=1)

    gthr = _gather(row, tbl)

    mrot = jnp.kron(W_rot, jnp.eye(4, dtype=jnp.float32))
    ws1e = jnp.broadcast_to(
        W_s1.reshape(_DEMB + _NSC, _H, _L, 1),
        (_DEMB + _NSC, _H, _L, 2)).reshape(_DEMB + _NSC, 4 * _H)
    wd = ws1e[:_DEMB]
    wx = ws1e[_DEMB:]
    be = jnp.broadcast_to(b_s1.reshape(_H, _L, 1),
                          (_H, _L, 2)).reshape(1, 4 * _H)
    m2 = jnp.einsum("hlo,cd->hlcod", W_out,
                    jnp.eye(2, dtype=jnp.float32)).reshape(4 * _H, _DOUT)

    xout = _mlp(gthr, distance_embedding, wd, wx, be, mrot, m2)

    q = _BE // 4
    colr = col.reshape(_E // _BE, 4, q).transpose(0, 2, 1).reshape(_E)

    zeros = jnp.zeros((_N, _DOUT), jnp.float32)
    parts = _scatter(colr, xout.reshape(_E, _DOUT), zeros)
    mess = _combine(parts[0], parts[1])
    return (x_scalar, mess.reshape(_N, _R, 2))


# speedup vs baseline: 5.5598x; 1.1006x over previous
import functools

import jax
import jax.numpy as jnp
from jax import lax
from jax.experimental import pallas as pl
from jax.experimental.pallas import tpu as pltpu
from jax.experimental.pallas import tpu_sc as plsc

_N = 50000
_E = 800000
_R = 16
_L = 2
_H = 3 * _R
_DEMB = 16
_NSC = 16
_DROT = 2 * _L * _R
_DOUT = 2 * _R
_DT = 128

_NC = 2
_NS = 16
_NW = _NC * _NS

_CHUNK = 128
_G_CHUNKS = _E // _CHUNK
_G_ITERS = -(-_G_CHUNKS // _NW)
_E_PER_CORE = _E // _NC
_S_CHUNKS = _E_PER_CORE // _CHUNK
_S_ITERS = -(-_S_CHUNKS // _NS)
_ROWS_PER_TILE = _N // _NS

_mesh = plsc.VectorSubcoreMesh(core_axis_name="c", subcore_axis_name="s")


@functools.partial(
    pl.kernel,
    out_type=jax.ShapeDtypeStruct((_E, _DT), jnp.float32),
    mesh=_mesh,
    scratch_types=[
        pltpu.VMEM((_CHUNK,), jnp.int32),
        pltpu.VMEM((_CHUNK, _DT), jnp.float32),
        pltpu.SemaphoreType.DMA,
    ],
)
def _gather(row_hbm, tbl_hbm, out_hbm, idx_v, rows_v, sem):
    wid = lax.axis_index("s") * _NC + lax.axis_index("c")

    def body(j, carry):
        cid = j * _NW + wid

        @pl.when(cid < _G_CHUNKS)
        def _():
            base = cid * _CHUNK
            pltpu.sync_copy(row_hbm.at[pl.ds(base, _CHUNK)], idx_v)
            pltpu.async_copy(tbl_hbm.at[idx_v], rows_v, sem).wait()
            pltpu.sync_copy(rows_v, out_hbm.at[pl.ds(base, _CHUNK)])

        return carry

    lax.fori_loop(0, _G_ITERS, body, 0)


@functools.partial(
    pl.kernel,
    out_type=jax.ShapeDtypeStruct((_NC, _N, _DOUT), jnp.float32),
    mesh=_mesh,
    scratch_types=[
        pltpu.VMEM((_CHUNK,), jnp.int32),
        pltpu.VMEM((_CHUNK, _DOUT), jnp.float32),
        pltpu.VMEM_SHARED((_N, _DOUT), jnp.float32),
    ],
    compiler_params=pltpu.CompilerParams(use_tc_tiling_on_sc=False),
)
def _scatter(col_hbm, xout_hbm, zero_hbm, part_hbm, idx_v, rows_v, acc):
    c = lax.axis_index("c")
    s = lax.axis_index("s")
    r0 = s * _ROWS_PER_TILE
    pltpu.sync_copy(zero_hbm.at[pl.ds(r0, _ROWS_PER_TILE)],
                    acc.at[pl.ds(r0, _ROWS_PER_TILE)])
    plsc.subcore_barrier()

    def body(j, carry):
        lcid = j * _NS + s

        @pl.when(lcid < _S_CHUNKS)
        def _():
            base = c * _E_PER_CORE + lcid * _CHUNK
            pltpu.sync_copy(col_hbm.at[pl.ds(base, _CHUNK)], idx_v)
            pltpu.sync_copy(xout_hbm.at[pl.ds(base, _CHUNK)], rows_v)
            pltpu.sync_copy(rows_v, acc.at[idx_v], add=True)

        return carry

    lax.fori_loop(0, _S_ITERS, body, 0)
    plsc.subcore_barrier()
    pltpu.sync_copy(acc.at[pl.ds(r0, _ROWS_PER_TILE)],
                    part_hbm.at[c, pl.ds(r0, _ROWS_PER_TILE)])


_BE = 4000


def _mlp_body(gthr_ref, demb_ref, wd_ref, wx_ref, b_ref,
              mrot_ref, m2_ref, out_ref):
    gt = gthr_ref[...].astype(jnp.bfloat16)
    xr = gt[:, :_DROT]
    xs = gt[:, _DROT:_DROT + _NSC]
    z = (jnp.dot(demb_ref[...].astype(jnp.bfloat16), wd_ref[...],
                 preferred_element_type=jnp.float32)
         + jnp.dot(xs, wx_ref[...], preferred_element_type=jnp.float32)
         + b_ref[...])
    g = z * jax.nn.sigmoid(z)
    h2 = jnp.dot(xr, mrot_ref[...], preferred_element_type=jnp.float32)
    res = jnp.dot((h2 * g).astype(jnp.bfloat16), m2_ref[...],
                  preferred_element_type=jnp.float32)
    q = _BE // 4
    out_ref[...] = jnp.concatenate(
        [res[0:q], res[q:2 * q], res[2 * q:3 * q], res[3 * q:4 * q]], axis=1)


_mlp = pl.pallas_call(
    _mlp_body,
    grid=(_E // _BE,),
    in_specs=[
        pl.BlockSpec((_BE, _DT), lambda i: (i, 0)),
        pl.BlockSpec((_BE, _DEMB), lambda i: (i, 0)),
        pl.BlockSpec((_DEMB, 4 * _H), lambda i: (0, 0)),
        pl.BlockSpec((_NSC, 4 * _H), lambda i: (0, 0)),
        pl.BlockSpec((1, 4 * _H), lambda i: (0, 0)),
        pl.BlockSpec((_DROT, 4 * _H), lambda i: (0, 0)),
        pl.BlockSpec((4 * _H, _DOUT), lambda i: (0, 0)),
    ],
    out_specs=pl.BlockSpec((_BE // 4, 128), lambda i: (i, 0)),
    out_shape=jax.ShapeDtypeStruct((_E // 4, 128), jnp.float32),
)


def _add_body(a_ref, b_ref, o_ref):
    o_ref[...] = a_ref[...] + b_ref[...]


_BN = 2000

_combine = pl.pallas_call(
    _add_body,
    grid=(_N // _BN,),
    in_specs=[
        pl.BlockSpec((_BN, _DOUT), lambda i: (i, 0)),
        pl.BlockSpec((_BN, _DOUT), lambda i: (i, 0)),
    ],
    out_specs=pl.BlockSpec((_BN, _DOUT), lambda i: (i, 0)),
    out_shape=jax.ShapeDtypeStruct((_N, _DOUT), jnp.float32),
)


def kernel(x_scalar, x_rot, edge_index, distance_embedding, rot,
           W_rot, W_s1, b_s1, W_out):
    del rot
    row = edge_index[0]
    col = edge_index[1]
    tbl = jnp.concatenate(
        [x_rot.reshape(_N, _DROT), x_scalar,
         jnp.zeros((_N, _DT - _DROT - _NSC), jnp.float32)], axis=1)

    gthr = _gather(row, tbl)

    mrot = jnp.kron(W_rot, jnp.eye(4, dtype=jnp.float32))
    ws1e = jnp.broadcast_to(
        W_s1.reshape(_DEMB + _NSC, _H, _L, 1),
        (_DEMB + _NSC, _H, _L, 2)).reshape(_DEMB + _NSC, 4 * _H)
    wd = ws1e[:_DEMB]
    wx = ws1e[_DEMB:]
    be = jnp.broadcast_to(b_s1.reshape(_H, _L, 1),
                          (_H, _L, 2)).reshape(1, 4 * _H)
    m2 = jnp.einsum("hlo,cd->hlcod", W_out,
                    jnp.eye(2, dtype=jnp.float32)).reshape(4 * _H, _DOUT)

    xout = _mlp(gthr, distance_embedding,
                wd.astype(jnp.bfloat16), wx.astype(jnp.bfloat16), be,
                mrot.astype(jnp.bfloat16), m2.astype(jnp.bfloat16))

    q = _BE // 4
    colr = col.reshape(_E // _BE, 4, q).transpose(0, 2, 1).reshape(_E)

    zeros = jnp.zeros((_N, _DOUT), jnp.float32)
    parts = _scatter(colr, xout.reshape(_E, _DOUT), zeros)
    mess = _combine(parts[0], parts[1])
    return (x_scalar, mess.reshape(_N, _R, 2))


# speedup vs baseline: 5.8350x; 1.0495x over previous
import functools

import jax
import jax.numpy as jnp
from jax import lax
from jax.experimental import pallas as pl
from jax.experimental.pallas import tpu as pltpu
from jax.experimental.pallas import tpu_sc as plsc

_N = 50000
_E = 800000
_R = 16
_L = 2
_H = 3 * _R
_DEMB = 16
_NSC = 16
_DROT = 2 * _L * _R
_DOUT = 2 * _R
_DT = 128

_NC = 2
_NS = 16
_NW = _NC * _NS

_CHUNK = 128
_G_CHUNKS = _E // _CHUNK
_G_ITERS = -(-_G_CHUNKS // _NW)
_E_PER_CORE = _E // _NC
_S_CHUNKS = _E_PER_CORE // _CHUNK
_S_ITERS = -(-_S_CHUNKS // _NS)
_ROWS_PER_TILE = _N // _NS

_mesh = plsc.VectorSubcoreMesh(core_axis_name="c", subcore_axis_name="s")


@functools.partial(
    pl.kernel,
    out_type=jax.ShapeDtypeStruct((_E, _DT), jnp.float32),
    mesh=_mesh,
    scratch_types=[
        pltpu.VMEM((_CHUNK,), jnp.int32),
        pltpu.VMEM((_CHUNK, _DT), jnp.float32),
        pltpu.SemaphoreType.DMA,
    ],
)
def _gather(row_hbm, tbl_hbm, out_hbm, idx_v, rows_v, sem):
    wid = lax.axis_index("s") * _NC + lax.axis_index("c")

    def body(j, carry):
        cid = j * _NW + wid

        @pl.when(cid < _G_CHUNKS)
        def _():
            base = cid * _CHUNK
            pltpu.sync_copy(row_hbm.at[pl.ds(base, _CHUNK)], idx_v)
            pltpu.async_copy(tbl_hbm.at[idx_v], rows_v, sem).wait()
            pltpu.sync_copy(rows_v, out_hbm.at[pl.ds(base, _CHUNK)])

        return carry

    lax.fori_loop(0, _G_ITERS, body, 0)


@functools.partial(
    pl.kernel,
    out_type=jax.ShapeDtypeStruct((_NC, _N, _DOUT), jnp.float32),
    mesh=_mesh,
    scratch_types=[
        pltpu.VMEM((_CHUNK,), jnp.int32),
        pltpu.VMEM((_CHUNK, _DOUT), jnp.float32),
        pltpu.VMEM_SHARED((_N, _DOUT), jnp.float32),
    ],
    compiler_params=pltpu.CompilerParams(use_tc_tiling_on_sc=False),
)
def _scatter(col_hbm, xout_hbm, zero_hbm, part_hbm, idx_v, rows_v, acc):
    c = lax.axis_index("c")
    s = lax.axis_index("s")
    r0 = s * _ROWS_PER_TILE
    pltpu.sync_copy(zero_hbm.at[pl.ds(r0, _ROWS_PER_TILE)],
                    acc.at[pl.ds(r0, _ROWS_PER_TILE)])
    plsc.subcore_barrier()

    def body(j, carry):
        lcid = j * _NS + s

        @pl.when(lcid < _S_CHUNKS)
        def _():
            base = c * _E_PER_CORE + lcid * _CHUNK
            pltpu.sync_copy(col_hbm.at[pl.ds(base, _CHUNK)], idx_v)
            pltpu.sync_copy(xout_hbm.at[pl.ds(base, _CHUNK)], rows_v)
            pltpu.sync_copy(rows_v, acc.at[idx_v], add=True)

        return carry

    lax.fori_loop(0, _S_ITERS, body, 0)
    plsc.subcore_barrier()
    pltpu.sync_copy(acc.at[pl.ds(r0, _ROWS_PER_TILE)],
                    part_hbm.at[c, pl.ds(r0, _ROWS_PER_TILE)])


_BE = 4000


def _mlp_body(gthr_ref, demb_ref, wxd_ref, b_ref, mrot_ref, m2_ref, out_ref):
    gt = gthr_ref[...].astype(jnp.bfloat16)
    xr = gt[:, :_DROT]
    sc = jnp.concatenate(
        [gt[:, _DROT:_DROT + _NSC],
         demb_ref[...].astype(jnp.bfloat16)], axis=1)
    z = (jnp.dot(sc, wxd_ref[...], preferred_element_type=jnp.float32)
         + b_ref[...])
    g = z * jax.nn.sigmoid(z)
    h2 = jnp.dot(xr, mrot_ref[...], preferred_element_type=jnp.float32)
    res = jnp.dot((h2 * g).astype(jnp.bfloat16), m2_ref[...],
                  preferred_element_type=jnp.float32)
    q = _BE // 4
    out_ref[...] = jnp.concatenate(
        [res[0:q], res[q:2 * q], res[2 * q:3 * q], res[3 * q:4 * q]], axis=1)


_mlp = pl.pallas_call(
    _mlp_body,
    grid=(_E // _BE,),
    in_specs=[
        pl.BlockSpec((_BE, _DT), lambda i: (i, 0)),
        pl.BlockSpec((_BE, _DEMB), lambda i: (i, 0)),
        pl.BlockSpec((_NSC + _DEMB, 4 * _H), lambda i: (0, 0)),
        pl.BlockSpec((1, 4 * _H), lambda i: (0, 0)),
        pl.BlockSpec((_DROT, 4 * _H), lambda i: (0, 0)),
        pl.BlockSpec((4 * _H, _DOUT), lambda i: (0, 0)),
    ],
    out_specs=pl.BlockSpec((_BE // 4, 128), lambda i: (i, 0)),
    out_shape=jax.ShapeDtypeStruct((_E // 4, 128), jnp.float32),
)


def _add_body(a_ref, b_ref, o_ref):
    o_ref[...] = a_ref[...] + b_ref[...]


_BN = 2000

_combine = pl.pallas_call(
    _add_body,
    grid=(_N // _BN,),
    in_specs=[
        pl.BlockSpec((_BN, _DOUT), lambda i: (i, 0)),
        pl.BlockSpec((_BN, _DOUT), lambda i: (i, 0)),
    ],
    out_specs=pl.BlockSpec((_BN, _DOUT), lambda i: (i, 0)),
    out_shape=jax.ShapeDtypeStruct((_N, _DOUT), jnp.float32),
)


def kernel(x_scalar, x_rot, edge_index, distance_embedding, rot,
           W_rot, W_s1, b_s1, W_out):
    del rot
    row = edge_index[0]
    col = edge_index[1]
    tbl = jnp.concatenate(
        [x_rot.reshape(_N, _DROT), x_scalar,
         jnp.zeros((_N, _DT - _DROT - _NSC), jnp.float32)], axis=1)

    gthr = _gather(row, tbl)

    mrot = jnp.kron(W_rot, jnp.eye(4, dtype=jnp.float32))
    ws1e = jnp.broadcast_to(
        W_s1.reshape(_DEMB + _NSC, _H, _L, 1),
        (_DEMB + _NSC, _H, _L, 2)).reshape(_DEMB + _NSC, 4 * _H)
    be = jnp.broadcast_to(b_s1.reshape(_H, _L, 1),
                          (_H, _L, 2)).reshape(1, 4 * _H)
    m2 = jnp.einsum("hlo,cd->hlcod", W_out,
                    jnp.eye(2, dtype=jnp.float32)).reshape(4 * _H, _DOUT)

    wxd = jnp.concatenate([ws1e[_DEMB:], ws1e[:_DEMB]], axis=0)
    xout = _mlp(gthr, distance_embedding, wxd.astype(jnp.bfloat16), be,
                mrot.astype(jnp.bfloat16), m2.astype(jnp.bfloat16))

    q = _BE // 4
    colr = col.reshape(_E // _BE, 4, q).transpose(0, 2, 1).reshape(_E)

    zeros = jnp.zeros((_N, _DOUT), jnp.float32)
    parts = _scatter(colr, xout.reshape(_E, _DOUT), zeros)
    mess = _combine(parts[0], parts[1])
    return (x_scalar, mess.reshape(_N, _R, 2))


# speedup vs baseline: 6.4988x; 1.1138x over previous
import functools

import jax
import jax.numpy as jnp
from jax import lax
from jax.experimental import pallas as pl
from jax.experimental.pallas import tpu as pltpu
from jax.experimental.pallas import tpu_sc as plsc

_N = 50000
_E = 800000
_R = 16
_L = 2
_H = 3 * _R
_DEMB = 16
_NSC = 16
_DROT = 2 * _L * _R
_DOUT = 2 * _R
_DT = 128

_NC = 2
_NS = 16
_NW = _NC * _NS

_CHUNK = 128
_G_CHUNKS = _E // _CHUNK
_G_ITERS = -(-_G_CHUNKS // _NW)
_E_PER_CORE = _E // _NC
_S_CHUNKS = _E_PER_CORE // _CHUNK
_S_ITERS = -(-_S_CHUNKS // _NS)
_ROWS_PER_TILE = _N // _NS

_mesh = plsc.VectorSubcoreMesh(core_axis_name="c", subcore_axis_name="s")


@functools.partial(
    pl.kernel,
    out_type=jax.ShapeDtypeStruct((_E, _DT), jnp.float32),
    mesh=_mesh,
    scratch_types=[
        pltpu.VMEM((2, _CHUNK), jnp.int32),
        pltpu.VMEM((2, _CHUNK, _DT), jnp.float32),
        pltpu.SemaphoreType.DMA,
        pltpu.SemaphoreType.DMA,
    ],
)
def _gather(row_hbm, tbl_hbm, out_hbm, idx_v, rows_v, sem0, sem1):
    wid = lax.axis_index("s") * _NC + lax.axis_index("c")
    sems = (sem0, sem1)

    def start(b, cid):
        pltpu.sync_copy(row_hbm.at[pl.ds(cid * _CHUNK, _CHUNK)], idx_v.at[b])
        pltpu.async_copy(tbl_hbm.at[idx_v.at[b]], rows_v.at[b], sems[b])

    def wait(b):
        pltpu.make_async_copy(tbl_hbm.at[idx_v.at[b]], rows_v.at[b],
                              sems[b]).wait()

    start(0, wid)

    def outer(t, carry):
        for b in (0, 1):
            j = 2 * t + b
            cidn = (j + 1) * _NW + wid

            @pl.when(cidn < _G_CHUNKS)
            def _():
                start(1 - b, cidn)

            cid = j * _NW + wid

            @pl.when(cid < _G_CHUNKS)
            def _():
                wait(b)
                pltpu.sync_copy(rows_v.at[b],
                                out_hbm.at[pl.ds(cid * _CHUNK, _CHUNK)])

        return carry

    lax.fori_loop(0, _G_ITERS // 2, outer, 0)


@functools.partial(
    pl.kernel,
    out_type=jax.ShapeDtypeStruct((_NC, _N, _DOUT), jnp.float32),
    mesh=_mesh,
    scratch_types=[
        pltpu.VMEM((2, _CHUNK), jnp.int32),
        pltpu.VMEM((2, _CHUNK, _DOUT), jnp.float32),
        pltpu.VMEM_SHARED((_N, _DOUT), jnp.float32),
        pltpu.SemaphoreType.DMA,
        pltpu.SemaphoreType.DMA,
    ],
    compiler_params=pltpu.CompilerParams(use_tc_tiling_on_sc=False),
)
def _scatter(col_hbm, xout_hbm, zero_hbm, part_hbm, idx_v, rows_v, acc,
             sem0, sem1):
    c = lax.axis_index("c")
    s = lax.axis_index("s")
    sems = (sem0, sem1)
    r0 = s * _ROWS_PER_TILE
    pltpu.sync_copy(zero_hbm.at[pl.ds(r0, _ROWS_PER_TILE)],
                    acc.at[pl.ds(r0, _ROWS_PER_TILE)])
    plsc.subcore_barrier()

    def start(b, lcid):
        base = c * _E_PER_CORE + lcid * _CHUNK
        pltpu.sync_copy(col_hbm.at[pl.ds(base, _CHUNK)], idx_v.at[b])
        pltpu.async_copy(xout_hbm.at[pl.ds(base, _CHUNK)], rows_v.at[b],
                         sems[b])

    def wait(b, lcid):
        base = c * _E_PER_CORE + lcid * _CHUNK
        pltpu.make_async_copy(xout_hbm.at[pl.ds(base, _CHUNK)],
                              rows_v.at[b], sems[b]).wait()

    start(0, s)

    def outer(t, carry):
        for b in (0, 1):
            j = 2 * t + b
            lcidn = (j + 1) * _NS + s

            @pl.when(lcidn < _S_CHUNKS)
            def _():
                start(1 - b, lcidn)

            lcid = j * _NS + s

            @pl.when(lcid < _S_CHUNKS)
            def _():
                wait(b, lcid)
                pltpu.sync_copy(rows_v.at[b], acc.at[idx_v.at[b]], add=True)

        return carry

    lax.fori_loop(0, _S_ITERS // 2, outer, 0)
    plsc.subcore_barrier()
    pltpu.sync_copy(acc.at[pl.ds(r0, _ROWS_PER_TILE)],
                    part_hbm.at[c, pl.ds(r0, _ROWS_PER_TILE)])


_BE = 4000


def _mlp_body(gthr_ref, demb_ref, wxd_ref, b_ref, mrot_ref, m2_ref, out_ref):
    gt = gthr_ref[...].astype(jnp.bfloat16)
    xr = gt[:, :_DROT]
    sc = jnp.concatenate(
        [gt[:, _DROT:_DROT + _NSC],
         demb_ref[...].astype(jnp.bfloat16)], axis=1)
    z = (jnp.dot(sc, wxd_ref[...], preferred_element_type=jnp.float32)
         + b_ref[...])
    g = z * jax.nn.sigmoid(z)
    h2 = jnp.dot(xr, mrot_ref[...], preferred_element_type=jnp.float32)
    res = jnp.dot((h2 * g).astype(jnp.bfloat16), m2_ref[...],
                  preferred_element_type=jnp.float32)
    q = _BE // 4
    out_ref[...] = jnp.concatenate(
        [res[0:q], res[q:2 * q], res[2 * q:3 * q], res[3 * q:4 * q]], axis=1)


_mlp = pl.pallas_call(
    _mlp_body,
    grid=(_E // _BE,),
    in_specs=[
        pl.BlockSpec((_BE, _DT), lambda i: (i, 0)),
        pl.BlockSpec((_BE, _DEMB), lambda i: (i, 0)),
        pl.BlockSpec((_NSC + _DEMB, 4 * _H), lambda i: (0, 0)),
        pl.BlockSpec((1, 4 * _H), lambda i: (0, 0)),
        pl.BlockSpec((_DROT, 4 * _H), lambda i: (0, 0)),
        pl.BlockSpec((4 * _H, _DOUT), lambda i: (0, 0)),
    ],
    out_specs=pl.BlockSpec((_BE // 4, 128), lambda i: (i, 0)),
    out_shape=jax.ShapeDtypeStruct((_E // 4, 128), jnp.float32),
)


def _add_body(a_ref, b_ref, o_ref):
    o_ref[...] = a_ref[...] + b_ref[...]


_BN = 2000

_combine = pl.pallas_call(
    _add_body,
    grid=(_N // _BN,),
    in_specs=[
        pl.BlockSpec((_BN, _DOUT), lambda i: (i, 0)),
        pl.BlockSpec((_BN, _DOUT), lambda i: (i, 0)),
    ],
    out_specs=pl.BlockSpec((_BN, _DOUT), lambda i: (i, 0)),
    out_shape=jax.ShapeDtypeStruct((_N, _DOUT), jnp.float32),
)


def kernel(x_scalar, x_rot, edge_index, distance_embedding, rot,
           W_rot, W_s1, b_s1, W_out):
    del rot
    row = edge_index[0]
    col = edge_index[1]
    tbl = jnp.concatenate(
        [x_rot.reshape(_N, _DROT), x_scalar,
         jnp.zeros((_N, _DT - _DROT - _NSC), jnp.float32)], axis=1)

    gthr = _gather(row, tbl)

    mrot = jnp.kron(W_rot, jnp.eye(4, dtype=jnp.float32))
    ws1e = jnp.broadcast_to(
        W_s1.reshape(_DEMB + _NSC, _H, _L, 1),
        (_DEMB + _NSC, _H, _L, 2)).reshape(_DEMB + _NSC, 4 * _H)
    be = jnp.broadcast_to(b_s1.reshape(_H, _L, 1),
                          (_H, _L, 2)).reshape(1, 4 * _H)
    m2 = jnp.einsum("hlo,cd->hlcod", W_out,
                    jnp.eye(2, dtype=jnp.float32)).reshape(4 * _H, _DOUT)

    wxd = jnp.concatenate([ws1e[_DEMB:], ws1e[:_DEMB]], axis=0)
    xout = _mlp(gthr, distance_embedding, wxd.astype(jnp.bfloat16), be,
                mrot.astype(jnp.bfloat16), m2.astype(jnp.bfloat16))

    q = _BE // 4
    colr = col.reshape(_E // _BE, 4, q).transpose(0, 2, 1).reshape(_E)

    zeros = jnp.zeros((_N, _DOUT), jnp.float32)
    parts = _scatter(colr, xout.reshape(_E, _DOUT), zeros)
    mess = _combine(parts[0], parts[1])
    return (x_scalar, mess.reshape(_N, _R, 2))
